# baseline (speedup 1.0000x reference)
import functools

import jax
import jax.numpy as jnp
from jax import lax
from jax.experimental import pallas as pl
from jax.experimental.pallas import tpu as pltpu
from jax.experimental.pallas import tpu_sc as plsc

_B = 1024
_D = 128
_KM = 100000
_TK = 5
_KB = 2000
_KBP = 2048
_G = 8
_GPB = _KBP // _G
_NCAND = _TK * _G
_CPB = 4
_NSTEP3 = _NCAND // _CPB
_NEG = float("-inf")
_BIG = 1 << 30
_FBIG = 1e9

_NC = 2
_NS = 16
_NW = _NC * _NS


def _gtopk_body(xt_ref, bank_ref, out_ref, xn_ref, scr_ref, rv_ref, ri_ref):
    step = pl.program_id(0)
    nsteps = pl.num_programs(0)
    nb = xt_ref.shape[1]

    @pl.when(step == 0)
    def _init():
        xv = xt_ref[...]
        ss = jnp.sum(xv * xv, axis=0, keepdims=True)
        xn_ref[...] = xv / (jnp.sqrt(ss) + 1e-8)
        scr_ref[_KB:, :] = jnp.full((_KBP - _KB, nb), _NEG, jnp.float32)
        rv_ref[...] = jnp.full((8, nb), _NEG, jnp.float32)
        ri_ref[...] = jnp.zeros((8, nb), jnp.float32)

    xn = xn_ref[...]
    base = step * _KBP
    iota_g = lax.broadcasted_iota(jnp.int32, (_GPB, nb), 0).astype(jnp.float32)

    if True:
        blk = bank_ref[...]
        ss = jnp.sum(blk * blk, axis=1, keepdims=True)
        bn = blk / (jnp.sqrt(ss) + 1e-8)
        scr_ref[:_KB, :] = jnp.dot(bn, xn,
                                   preferred_element_type=jnp.float32)
        t1 = jnp.maximum(scr_ref[:_KBP // 2, :], scr_ref[_KBP // 2:, :])
        t2 = jnp.maximum(t1[:_KBP // 4], t1[_KBP // 4:])
        mg = jnp.maximum(t2[:_GPB], t2[_GPB:])
        ig = jnp.full((_GPB, nb), _FBIG, jnp.float32)
        for j in range(_G):
            rid = iota_g + (base + _GPB * j).astype(jnp.float32)
            ig = jnp.minimum(
                ig, jnp.where(scr_ref[_GPB * j:_GPB * (j + 1), :] == mg, rid,
                              _FBIG))

        rv = rv_ref[...]
        ri = ri_ref[...]
        vals, idxs = [], []
        for _ in range(_TK):
            m = jnp.maximum(jnp.max(mg, axis=0, keepdims=True),
                            jnp.max(rv, axis=0, keepdims=True))
            f_mg = jnp.min(jnp.where(mg == m, ig, _FBIG), axis=0,
                           keepdims=True)
            f_run = jnp.min(jnp.where(rv == m, ri, _FBIG), axis=0,
                            keepdims=True)
            sel = jnp.minimum(f_mg, f_run)
            vals.append(m)
            idxs.append(sel)
            if len(vals) < _TK:
                mg = jnp.where(ig == sel, _NEG, mg)
                rv = jnp.where(ri == sel, _NEG, rv)
        new_rv = jnp.concatenate(
            vals + [jnp.full((8 - _TK, nb), _NEG, jnp.float32)], axis=0)
        new_ri = jnp.concatenate(
            idxs + [jnp.zeros((8 - _TK, nb), jnp.float32)], axis=0)
        rv_ref[...] = new_rv
        ri_ref[...] = new_ri

        @pl.when(step == nsteps - 1)
        def _emit():
            out_ref[...] = new_ri.astype(jnp.int32)


def _gtopk_call(xt, bank):
    km = bank.shape[0]
    nb = xt.shape[1]
    nsteps = km // _KB
    assert nsteps * _KB == km
    return pl.pallas_call(
        _gtopk_body,
        grid=(nsteps,),
        in_specs=[
            pl.BlockSpec((_D, nb), lambda i: (0, 0)),
            pl.BlockSpec((_KB, _D), lambda i: (i, 0)),
        ],
        out_specs=pl.BlockSpec((8, nb), lambda i: (0, 0)),
        out_shape=jax.ShapeDtypeStruct((8, nb), jnp.int32),
        scratch_shapes=[
            pltpu.VMEM((_D, nb), jnp.float32),
            pltpu.VMEM((_KBP, nb), jnp.float32),
            pltpu.VMEM((8, nb), jnp.float32),
            pltpu.VMEM((8, nb), jnp.float32),
        ],
        compiler_params=pltpu.CompilerParams(
            dimension_semantics=("arbitrary",)),
    )(xt, bank)


def _sc_gather_cands(bank, f1d):
    nrows = f1d.shape[0]
    chunk = 80
    r_per_w = nrows // _NW
    nchunk = r_per_w // chunk
    mesh = plsc.VectorSubcoreMesh(core_axis_name="c", subcore_axis_name="s")

    @functools.partial(
        pl.kernel,
        mesh=mesh,
        out_type=jax.ShapeDtypeStruct((nrows, _D), jnp.float32),
        scratch_types=[
            pltpu.VMEM((chunk,), jnp.int32),
            pltpu.VMEM((chunk,), jnp.int32),
            pltpu.VMEM((chunk, _D), jnp.float32),
            pltpu.VMEM((chunk, _D), jnp.float32),
            pltpu.SemaphoreType.DMA,
            pltpu.SemaphoreType.DMA,
            pltpu.SemaphoreType.DMA,
            pltpu.SemaphoreType.DMA,
            pltpu.SemaphoreType.DMA,
            pltpu.SemaphoreType.DMA,
        ],
    )
    def gk(bank_h, idx_h, out_h, idx_a, idx_b, rows_a, rows_b,
           isem_a, isem_b, gsem_a, gsem_b, wsem_a, wsem_b):
        cid = lax.axis_index("c")
        sid = lax.axis_index("s")
        wid = sid * _NC + cid
        rbase = wid * r_per_w
        idxb = (idx_a, idx_b)
        rows = (rows_a, rows_b)
        isems = (isem_a, isem_b)
        gsems = (gsem_a, gsem_b)
        wsems = (wsem_a, wsem_b)
        if True:
            ipend = [None, None]
            gpend = [None, None]
            wpend = [None, None]
            for h in range(min(2, nchunk)):
                ipend[h] = pltpu.async_copy(
                    idx_h.at[pl.ds(rbase + h * chunk, chunk)], idxb[h],
                    isems[h])
            for h in range(nchunk):
                b = h % 2
                if wpend[b] is not None:
                    wpend[b].wait()
                ipend[b].wait()
                gpend[b] = pltpu.async_copy(bank_h.at[idxb[b]], rows[b],
                                            gsems[b])
                if h > 0:
                    bp = 1 - b
                    gpend[bp].wait()
                    wpend[bp] = pltpu.async_copy(
                        rows[bp],
                        out_h.at[pl.ds(rbase + (h - 1) * chunk, chunk)],
                        wsems[bp])
                    if h + 1 < nchunk:
                        ipend[bp] = pltpu.async_copy(
                            idx_h.at[pl.ds(rbase + (h + 1) * chunk, chunk)],
                            idxb[bp], isems[bp])
            bl = (nchunk - 1) % 2
            gpend[bl].wait()
            wpend[bl] = pltpu.async_copy(
                rows[bl],
                out_h.at[pl.ds(rbase + (nchunk - 1) * chunk, chunk)],
                wsems[bl])
            if wpend[1 - bl] is not None:
                wpend[1 - bl].wait()
            wpend[bl].wait()

    return gk(bank, f1d)


def _rescore_body(gp_ref, gn_ref, x_ref, idsp_ref, idsn_ref, wf_ref, bf_ref,
                  w1_ref, b1_ref, w2_ref, b2_ref, o_ref,
                  xn_ref, sp_ref, sn_ref, mp_ref, mn_ref, ap_ref, an_ref):
    i = pl.program_id(0)
    nb = x_ref.shape[0]
    lane = lax.broadcasted_iota(jnp.int32, (1, _D), 1)

    @pl.when(i == 0)
    def _init():
        xv = x_ref[...]
        ss = jnp.sum(xv * xv, axis=1, keepdims=True)
        xn_ref[...] = xv / (jnp.sqrt(ss) + 1e-8)
        zf = jnp.zeros((nb, _D), jnp.float32)
        sp_ref[...] = zf
        sn_ref[...] = zf
        ap_ref[...] = zf
        an_ref[...] = zf

    @pl.when(i < _NSTEP3)
    def _score():
        xn = xn_ref[...]
        ones_col = jnp.ones((_D, 1), jnp.float32)
        for g_ref, s_ref in ((gp_ref, sp_ref), (gn_ref, sn_ref)):
            acc = jnp.zeros((nb, _D), jnp.float32)
            for sub in range(_CPB):
                c = i * _CPB + sub
                oneh = (lane == c).astype(jnp.float32)
                g = g_ref[sub * nb:(sub + 1) * nb, :]
                rn = jnp.sqrt(
                    jnp.dot(g * g, ones_col,
                            preferred_element_type=jnp.float32)) + 1e-8
                s_col = jnp.dot(g * xn, ones_col,
                                preferred_element_type=jnp.float32) / rn
                acc = acc + s_col * oneh
            s_ref[...] += acc

    @pl.when(i == _NSTEP3)
    def _select():
        for s_ref, ids_ref, m_ref in ((sp_ref, idsp_ref, mp_ref),
                                      (sn_ref, idsn_ref, mn_ref)):
            ids = ids_ref[...]
            s = jnp.where(ids < _BIG, s_ref[...], _NEG)
            mask = jnp.zeros((nb, _D), jnp.float32)
            for _ in range(_TK):
                m = jnp.max(s, axis=1, keepdims=True)
                sel = jnp.min(jnp.where(s == m, ids, _BIG), axis=1,
                              keepdims=True)
                hit = (ids == sel)
                mask = mask + hit.astype(jnp.float32)
                s = jnp.where(hit, _NEG, s)
            m_ref[...] = mask

    @pl.when(i >= _NSTEP3)
    def _accum():
        for g_ref, m_ref, a_ref in ((gp_ref, mp_ref, ap_ref),
                                    (gn_ref, mn_ref, an_ref)):
            mval = m_ref[...]
            acc = jnp.zeros((nb, _D), jnp.float32)
            for sub in range(_CPB):
                col = mval[:, 0:1]
                acc = acc + g_ref[sub * nb:(sub + 1) * nb, :] * col
                mval = pltpu.roll(mval, _D - 1, 1)
            a_ref[...] += acc
            m_ref[...] = mval

    @pl.when(i == 2 * _NSTEP3 - 1)
    def _head():
        xv = x_ref[...]
        pv = ap_ref[...] / jnp.float32(_TK)
        nv = an_ref[...] / jnp.float32(_TK)
        wf = wf_ref[...]
        a = (jnp.dot(xv, wf[0:_D], preferred_element_type=jnp.float32)
             + jnp.dot(pv, wf[_D:2 * _D], preferred_element_type=jnp.float32)
             + jnp.dot(nv, wf[2 * _D:3 * _D],
                       preferred_element_type=jnp.float32)
             + bf_ref[...])
        calib = xv + jnp.tanh(a)
        h = jnp.maximum(
            jnp.dot(calib, w1_ref[...], preferred_element_type=jnp.float32)
            + b1_ref[...], 0.0)
        o_ref[...] = (jnp.dot(h, w2_ref[...],
                              preferred_element_type=jnp.float32)
                      + b2_ref[...])


def _rescore_call(gp, gn, x, idsp, idsn, w_fuse, bf, w1p, b1p, w2p, b2p):
    nb = x.shape[0]
    cand_map = lambda i: (i % _NSTEP3, 0)
    const_map = lambda i: (0, 0)
    return pl.pallas_call(
        _rescore_body,
        grid=(2 * _NSTEP3,),
        in_specs=[
            pl.BlockSpec((_CPB * nb, _D), cand_map),
            pl.BlockSpec((_CPB * nb, _D), cand_map),
            pl.BlockSpec((nb, _D), const_map),
            pl.BlockSpec((nb, _D), const_map),
            pl.BlockSpec((nb, _D), const_map),
            pl.BlockSpec((3 * _D, _D), const_map),
            pl.BlockSpec((1, _D), const_map),
            pl.BlockSpec((_D, _D), const_map),
            pl.BlockSpec((1, _D), const_map),
            pl.BlockSpec((_D, _D), const_map),
            pl.BlockSpec((1, _D), const_map),
        ],
        out_specs=pl.BlockSpec((nb, _D), const_map),
        out_shape=jax.ShapeDtypeStruct((nb, _D), jnp.float32),
        scratch_shapes=[pltpu.VMEM((nb, _D), jnp.float32) for _ in range(7)],
        compiler_params=pltpu.CompilerParams(
            dimension_semantics=("arbitrary",)),
    )(gp, gn, x, idsp, idsn, w_fuse, bf, w1p, b1p, w2p, b2p)


def kernel(x, pos_bank, neg_bank, W_fuse, b_fuse, W1, b1, W2, b2):
    nb, d = x.shape
    xt = x.T

    gp8 = _gtopk_call(xt, pos_bank)
    gn8 = _gtopk_call(xt, neg_bank)

    def _expand(a8):
        aid = a8[:_TK].T
        stp = aid // _KBP
        g = (aid % _KBP) % _GPB
        off = jnp.arange(_G, dtype=jnp.int32) * _GPB
        local = (g[:, :, None] + off).reshape(nb, _NCAND)
        stp = jnp.repeat(stp, _G, axis=1)
        valid = local < _KB
        ids = jnp.where(valid, stp * _KBP + local, _BIG)
        grow = jnp.where(valid, stp * _KB + local, 0)
        return ids, grow

    idsp_c, gp_rows = _expand(gp8)
    idsn_c, gn_rows = _expand(gn8)
    fp2 = gp_rows.T.reshape(-1)
    fn2 = gn_rows.T.reshape(-1)
    idsp = jnp.pad(idsp_c, ((0, 0), (0, _D - _NCAND)), constant_values=_BIG)
    idsn = jnp.pad(idsn_c, ((0, 0), (0, _D - _NCAND)), constant_values=_BIG)

    gpr = _sc_gather_cands(pos_bank, fp2)
    gnr = _sc_gather_cands(neg_bank, fn2)

    w1p = jnp.pad(W1, ((0, 0), (0, _D - W1.shape[1])))
    b1p = jnp.pad(b1, (0, _D - b1.shape[0])).reshape(1, _D)
    w2p = jnp.pad(W2, ((0, _D - W2.shape[0]), (0, _D - W2.shape[1])))
    b2p = jnp.pad(b2, (0, _D - b2.shape[0])).reshape(1, _D)
    bf = b_fuse.reshape(1, _D)

    out = _rescore_call(gpr, gnr, x, idsp, idsn, W_fuse, bf, w1p, b1p, w2p,
                        b2p)
    return out[:, :W2.shape[1]]


# speedup vs baseline: 1.0240x; 1.0240x over previous
import functools

import jax
import jax.numpy as jnp
from jax import lax
from jax.experimental import pallas as pl
from jax.experimental.pallas import tpu as pltpu
from jax.experimental.pallas import tpu_sc as plsc

_B = 1024
_D = 128
_KM = 100000
_TK = 5
_KB = 2000
_KBP = 2048
_G = 8
_GPB = _KBP // _G
_NCAND = _TK * _G
_CPB = 4
_NSTEP3 = _NCAND // _CPB
_NEG = float("-inf")
_BIG = 1 << 30
_FBIG = 1e9

_NC = 2
_NS = 16
_NW = _NC * _NS


def _gtopk_body(xt_ref, pos_ref, neg_ref, gp_ref, gn_ref,
                xn_ref, scr_ref, rvp_ref, rip_ref, rvn_ref, rin_ref):
    step = pl.program_id(0)
    nsteps = pl.num_programs(0)
    nb = xt_ref.shape[1]

    @pl.when(step == 0)
    def _init():
        xv = xt_ref[...]
        ss = jnp.sum(xv * xv, axis=0, keepdims=True)
        xn_ref[...] = xv / (jnp.sqrt(ss) + 1e-8)
        scr_ref[_KB:, :] = jnp.full((_KBP - _KB, nb), _NEG, jnp.float32)
        neg_fill = jnp.full((8, nb), _NEG, jnp.float32)
        zf = jnp.zeros((8, nb), jnp.float32)
        rvp_ref[...] = neg_fill
        rvn_ref[...] = neg_fill
        rip_ref[...] = zf
        rin_ref[...] = zf

    xn = xn_ref[...]
    base = step * _KBP
    iota_g = lax.broadcasted_iota(jnp.int32, (_GPB, nb), 0).astype(jnp.float32)

    for bank_ref, rv_ref, ri_ref, out_ref in (
            (pos_ref, rvp_ref, rip_ref, gp_ref),
            (neg_ref, rvn_ref, rin_ref, gn_ref)):
        blk = bank_ref[...]
        ss = jnp.sum(blk * blk, axis=1, keepdims=True)
        bn = blk / (jnp.sqrt(ss) + 1e-8)
        scr_ref[:_KB, :] = jnp.dot(bn, xn,
                                   preferred_element_type=jnp.float32)
        t1 = jnp.maximum(scr_ref[:_KBP // 2, :], scr_ref[_KBP // 2:, :])
        t2 = jnp.maximum(t1[:_KBP // 4], t1[_KBP // 4:])
        mg = jnp.maximum(t2[:_GPB], t2[_GPB:])
        ig = jnp.full((_GPB, nb), _FBIG, jnp.float32)
        for j in range(_G):
            rid = iota_g + (base + _GPB * j).astype(jnp.float32)
            ig = jnp.minimum(
                ig, jnp.where(scr_ref[_GPB * j:_GPB * (j + 1), :] == mg, rid,
                              _FBIG))

        rv = rv_ref[...]
        ri = ri_ref[...]
        vals, idxs = [], []
        for _ in range(_TK):
            m = jnp.maximum(jnp.max(mg, axis=0, keepdims=True),
                            jnp.max(rv, axis=0, keepdims=True))
            f_mg = jnp.min(jnp.where(mg == m, ig, _FBIG), axis=0,
                           keepdims=True)
            f_run = jnp.min(jnp.where(rv == m, ri, _FBIG), axis=0,
                            keepdims=True)
            sel = jnp.minimum(f_mg, f_run)
            vals.append(m)
            idxs.append(sel)
            if len(vals) < _TK:
                mg = jnp.where(ig == sel, _NEG, mg)
                rv = jnp.where(ri == sel, _NEG, rv)
        new_rv = jnp.concatenate(
            vals + [jnp.full((8 - _TK, nb), _NEG, jnp.float32)], axis=0)
        new_ri = jnp.concatenate(
            idxs + [jnp.zeros((8 - _TK, nb), jnp.float32)], axis=0)
        rv_ref[...] = new_rv
        ri_ref[...] = new_ri

        @pl.when(step == nsteps - 1)
        def _emit():
            out_ref[...] = new_ri.astype(jnp.int32)


def _gtopk_call(xt, pos_bank, neg_bank):
    km = pos_bank.shape[0]
    nb = xt.shape[1]
    nsteps = km // _KB
    assert nsteps * _KB == km
    return pl.pallas_call(
        _gtopk_body,
        grid=(nsteps,),
        in_specs=[
            pl.BlockSpec((_D, nb), lambda i: (0, 0)),
            pl.BlockSpec((_KB, _D), lambda i: (i, 0)),
            pl.BlockSpec((_KB, _D), lambda i: (i, 0)),
        ],
        out_specs=[
            pl.BlockSpec((8, nb), lambda i: (0, 0)),
            pl.BlockSpec((8, nb), lambda i: (0, 0)),
        ],
        out_shape=[
            jax.ShapeDtypeStruct((8, nb), jnp.int32),
            jax.ShapeDtypeStruct((8, nb), jnp.int32),
        ],
        scratch_shapes=[
            pltpu.VMEM((_D, nb), jnp.float32),
            pltpu.VMEM((_KBP, nb), jnp.float32),
            pltpu.VMEM((8, nb), jnp.float32),
            pltpu.VMEM((8, nb), jnp.float32),
            pltpu.VMEM((8, nb), jnp.float32),
            pltpu.VMEM((8, nb), jnp.float32),
        ],
        compiler_params=pltpu.CompilerParams(
            dimension_semantics=("arbitrary",)),
    )(xt, pos_bank, neg_bank)


def _sc_gather_cands(pos_bank, neg_bank, fp2, fn2):
    nrows = fp2.shape[0]
    chunk = 80
    r_per_w = nrows // _NW
    nchunk = r_per_w // chunk
    mesh = plsc.VectorSubcoreMesh(core_axis_name="c", subcore_axis_name="s")

    @functools.partial(
        pl.kernel,
        mesh=mesh,
        out_type=[
            jax.ShapeDtypeStruct((nrows, _D), jnp.float32),
            jax.ShapeDtypeStruct((nrows, _D), jnp.float32),
        ],
        scratch_types=[
            pltpu.VMEM((chunk,), jnp.int32),
            pltpu.VMEM((chunk,), jnp.int32),
            pltpu.VMEM((chunk, _D), jnp.float32),
            pltpu.VMEM((chunk, _D), jnp.float32),
            pltpu.SemaphoreType.DMA,
            pltpu.SemaphoreType.DMA,
            pltpu.SemaphoreType.DMA,
            pltpu.SemaphoreType.DMA,
            pltpu.SemaphoreType.DMA,
            pltpu.SemaphoreType.DMA,
        ],
    )
    def gk(posb, negb, fph, fnh, gp, gn, idx_a, idx_b, rows_a, rows_b,
           isem_a, isem_b, gsem_a, gsem_b, wsem_a, wsem_b):
        cid = lax.axis_index("c")
        sid = lax.axis_index("s")
        wid = sid * _NC + cid
        rbase = wid * r_per_w
        idxb = (idx_a, idx_b)
        rows = (rows_a, rows_b)
        isems = (isem_a, isem_b)
        gsems = (gsem_a, gsem_b)
        wsems = (wsem_a, wsem_b)
        for bank_h, idx_h, out_h in ((posb, fph, gp), (negb, fnh, gn)):
            ipend = [None, None]
            gpend = [None, None]
            wpend = [None, None]
            for h in range(min(2, nchunk)):
                ipend[h] = pltpu.async_copy(
                    idx_h.at[pl.ds(rbase + h * chunk, chunk)], idxb[h],
                    isems[h])
            for h in range(nchunk):
                b = h % 2
                if wpend[b] is not None:
                    wpend[b].wait()
                ipend[b].wait()
                gpend[b] = pltpu.async_copy(bank_h.at[idxb[b]], rows[b],
                                            gsems[b])
                if h > 0:
                    bp = 1 - b
                    gpend[bp].wait()
                    wpend[bp] = pltpu.async_copy(
                        rows[bp],
                        out_h.at[pl.ds(rbase + (h - 1) * chunk, chunk)],
                        wsems[bp])
                    if h + 1 < nchunk:
                        ipend[bp] = pltpu.async_copy(
                            idx_h.at[pl.ds(rbase + (h + 1) * chunk, chunk)],
                            idxb[bp], isems[bp])
            bl = (nchunk - 1) % 2
            gpend[bl].wait()
            wpend[bl] = pltpu.async_copy(
                rows[bl],
                out_h.at[pl.ds(rbase + (nchunk - 1) * chunk, chunk)],
                wsems[bl])
            if wpend[1 - bl] is not None:
                wpend[1 - bl].wait()
            wpend[bl].wait()

    return gk(pos_bank, neg_bank, fp2, fn2)


def _rescore_body(gp_ref, gn_ref, x_ref, idsp_ref, idsn_ref, wf_ref, bf_ref,
                  w1_ref, b1_ref, w2_ref, b2_ref, o_ref,
                  xn_ref, sp_ref, sn_ref, mp_ref, mn_ref, ap_ref, an_ref):
    i = pl.program_id(0)
    nb = x_ref.shape[0]
    lane = lax.broadcasted_iota(jnp.int32, (1, _D), 1)

    @pl.when(i == 0)
    def _init():
        xv = x_ref[...]
        ss = jnp.sum(xv * xv, axis=1, keepdims=True)
        xn_ref[...] = xv / (jnp.sqrt(ss) + 1e-8)
        zf = jnp.zeros((nb, _D), jnp.float32)
        sp_ref[...] = zf
        sn_ref[...] = zf
        ap_ref[...] = zf
        an_ref[...] = zf

    @pl.when(i < _NSTEP3)
    def _score():
        xn = xn_ref[...]
        ones_col = jnp.ones((_D, 1), jnp.float32)
        for g_ref, s_ref in ((gp_ref, sp_ref), (gn_ref, sn_ref)):
            acc = jnp.zeros((nb, _D), jnp.float32)
            for sub in range(_CPB):
                c = i * _CPB + sub
                oneh = (lane == c).astype(jnp.float32)
                g = g_ref[sub * nb:(sub + 1) * nb, :]
                rn = jnp.sqrt(
                    jnp.dot(g * g, ones_col,
                            preferred_element_type=jnp.float32)) + 1e-8
                s_col = jnp.dot(g * xn, ones_col,
                                preferred_element_type=jnp.float32) / rn
                acc = acc + s_col * oneh
            s_ref[...] += acc

    @pl.when(i == _NSTEP3)
    def _select():
        for s_ref, ids_ref, m_ref in ((sp_ref, idsp_ref, mp_ref),
                                      (sn_ref, idsn_ref, mn_ref)):
            ids = ids_ref[...]
            s = jnp.where(ids < _BIG, s_ref[...], _NEG)
            mask = jnp.zeros((nb, _D), jnp.float32)
            for _ in range(_TK):
                m = jnp.max(s, axis=1, keepdims=True)
                sel = jnp.min(jnp.where(s == m, ids, _BIG), axis=1,
                              keepdims=True)
                hit = (ids == sel)
                mask = mask + hit.astype(jnp.float32)
                s = jnp.where(hit, _NEG, s)
            m_ref[...] = mask

    @pl.when(i >= _NSTEP3)
    def _accum():
        for g_ref, m_ref, a_ref in ((gp_ref, mp_ref, ap_ref),
                                    (gn_ref, mn_ref, an_ref)):
            mval = m_ref[...]
            acc = jnp.zeros((nb, _D), jnp.float32)
            for sub in range(_CPB):
                col = mval[:, 0:1]
                acc = acc + g_ref[sub * nb:(sub + 1) * nb, :] * col
                mval = pltpu.roll(mval, _D - 1, 1)
            a_ref[...] += acc
            m_ref[...] = mval

    @pl.when(i == 2 * _NSTEP3 - 1)
    def _head():
        xv = x_ref[...]
        pv = ap_ref[...] / jnp.float32(_TK)
        nv = an_ref[...] / jnp.float32(_TK)
        wf = wf_ref[...]
        a = (jnp.dot(xv, wf[0:_D], preferred_element_type=jnp.float32)
             + jnp.dot(pv, wf[_D:2 * _D], preferred_element_type=jnp.float32)
             + jnp.dot(nv, wf[2 * _D:3 * _D],
                       preferred_element_type=jnp.float32)
             + bf_ref[...])
        calib = xv + jnp.tanh(a)
        h = jnp.maximum(
            jnp.dot(calib, w1_ref[...], preferred_element_type=jnp.float32)
            + b1_ref[...], 0.0)
        o_ref[...] = (jnp.dot(h, w2_ref[...],
                              preferred_element_type=jnp.float32)
                      + b2_ref[...])


def _rescore_call(gp, gn, x, idsp, idsn, w_fuse, bf, w1p, b1p, w2p, b2p):
    nb = x.shape[0]
    cand_map = lambda i: (i % _NSTEP3, 0)
    const_map = lambda i: (0, 0)
    return pl.pallas_call(
        _rescore_body,
        grid=(2 * _NSTEP3,),
        in_specs=[
            pl.BlockSpec((_CPB * nb, _D), cand_map),
            pl.BlockSpec((_CPB * nb, _D), cand_map),
            pl.BlockSpec((nb, _D), const_map),
            pl.BlockSpec((nb, _D), const_map),
            pl.BlockSpec((nb, _D), const_map),
            pl.BlockSpec((3 * _D, _D), const_map),
            pl.BlockSpec((1, _D), const_map),
            pl.BlockSpec((_D, _D), const_map),
            pl.BlockSpec((1, _D), const_map),
            pl.BlockSpec((_D, _D), const_map),
            pl.BlockSpec((1, _D), const_map),
        ],
        out_specs=pl.BlockSpec((nb, _D), const_map),
        out_shape=jax.ShapeDtypeStruct((nb, _D), jnp.float32),
        scratch_shapes=[pltpu.VMEM((nb, _D), jnp.float32) for _ in range(7)],
        compiler_params=pltpu.CompilerParams(
            dimension_semantics=("arbitrary",)),
    )(gp, gn, x, idsp, idsn, w_fuse, bf, w1p, b1p, w2p, b2p)


def kernel(x, pos_bank, neg_bank, W_fuse, b_fuse, W1, b1, W2, b2):
    nb, d = x.shape
    xt = x.T

    gp8, gn8 = _gtopk_call(xt, pos_bank, neg_bank)

    def _expand(a8):
        aid = a8[:_TK].T
        stp = aid // _KBP
        g = (aid % _KBP) % _GPB
        off = jnp.arange(_G, dtype=jnp.int32) * _GPB
        local = (g[:, :, None] + off).reshape(nb, _NCAND)
        stp = jnp.repeat(stp, _G, axis=1)
        valid = local < _KB
        ids = jnp.where(valid, stp * _KBP + local, _BIG)
        grow = jnp.where(valid, stp * _KB + local, 0)
        return ids, grow

    idsp_c, gp_rows = _expand(gp8)
    idsn_c, gn_rows = _expand(gn8)
    fp2 = gp_rows.T.reshape(-1)
    fn2 = gn_rows.T.reshape(-1)
    idsp = jnp.pad(idsp_c, ((0, 0), (0, _D - _NCAND)), constant_values=_BIG)
    idsn = jnp.pad(idsn_c, ((0, 0), (0, _D - _NCAND)), constant_values=_BIG)

    gpr, gnr = _sc_gather_cands(pos_bank, neg_bank, fp2, fn2)

    w1p = jnp.pad(W1, ((0, 0), (0, _D - W1.shape[1])))
    b1p = jnp.pad(b1, (0, _D - b1.shape[0])).reshape(1, _D)
    w2p = jnp.pad(W2, ((0, _D - W2.shape[0]), (0, _D - W2.shape[1])))
    b2p = jnp.pad(b2, (0, _D - b2.shape[0])).reshape(1, _D)
    bf = b_fuse.reshape(1, _D)

    out = _rescore_call(gpr, gnr, x, idsp, idsn, W_fuse, bf, w1p, b1p, w2p,
                        b2p)
    return out[:, :W2.shape[1]]
